# Initial kernel scaffold; baseline (speedup 1.0000x reference)
import jax
import jax.numpy as jnp
from jax.experimental import pallas as pl


def kernel(center_feat, neighbor_feats, W1, W2):
    raise NotImplementedError("write your pallas kernel here")


# speedup vs baseline: 2.4788x; 2.4788x over previous
import functools

import jax
import jax.numpy as jnp
from jax.experimental import pallas as pl
from jax.experimental.pallas import tpu as pltpu

_R = 512


def _mlp_body(c_ref, n_ref, w1_ref, w2_ref, m_ref, o_ref, *, rows, S, D):
    c = c_ref[...]
    x = n_ref[...]
    w1 = w1_ref[...]
    w2 = w2_ref[...]
    sn = jnp.tanh(jax.lax.dot(x, w1, preferred_element_type=jnp.float32))
    pn = jnp.sum((sn * x).reshape(rows, S, D), axis=1)
    sc = jnp.tanh(jax.lax.dot(c, w1, preferred_element_type=jnp.float32))
    t = pn + sc * c
    agg = jax.lax.dot(t, w2, preferred_element_type=jnp.float32)
    m = m_ref[...]
    o_ref[...] = jnp.where(m > 0.0, c, agg)


def _mlp_all_rows(center_feat, neighbor_flat, W1, W2, is_neg, *, interpret=False):
    B, D = center_feat.shape
    S = neighbor_flat.shape[0] // B
    R = _R
    grid = (B // R,)
    body = functools.partial(_mlp_body, rows=R, S=S, D=D)
    return pl.pallas_call(
        body,
        grid=grid,
        in_specs=[
            pl.BlockSpec((R, D), lambda i: (i, 0)),
            pl.BlockSpec((R * S, D), lambda i: (i, 0)),
            pl.BlockSpec((D, D), lambda i: (0, 0)),
            pl.BlockSpec((D, D), lambda i: (0, 0)),
            pl.BlockSpec((R, 1), lambda i: (i, 0)),
        ],
        out_specs=pl.BlockSpec((R, D), lambda i: (i, 0)),
        out_shape=jax.ShapeDtypeStruct((B, D), jnp.float32),
        compiler_params=pltpu.CompilerParams(
            dimension_semantics=("arbitrary",),
        ),
        interpret=interpret,
    )(center_feat, neighbor_flat, W1, W2, is_neg)


def kernel(center_feat, neighbor_feats, W1, W2):
    B, D = center_feat.shape
    S = neighbor_feats.shape[1]
    ano = int(B * 0.1)
    batch_center = jnp.mean(center_feat, axis=-1)
    diff_center = jnp.sum(center_feat - batch_center[:, None], axis=-1)
    sorted_idx = jnp.argsort(diff_center)
    neg_idx = sorted_idx[B - ano:]
    is_neg = jnp.zeros((B,), jnp.float32).at[neg_idx].set(1.0)[:, None]
    out = _mlp_all_rows(center_feat, neighbor_feats.reshape(B * S, D), W1, W2,
                        is_neg)
    return out, neg_idx


# speedup vs baseline: 2.4859x; 1.0029x over previous
import functools

import jax
import jax.numpy as jnp
from jax.experimental import pallas as pl
from jax.experimental.pallas import tpu as pltpu

_R = 512


def _mlp_body(c_ref, n_ref, w1_ref, w2_ref, m_ref, o_ref, *, rows, S, D):
    c = c_ref[...]
    x = n_ref[...]
    w1 = w1_ref[...]
    w2 = w2_ref[...]
    sn = jnp.tanh(jax.lax.dot(x, w1, preferred_element_type=jnp.float32))
    pn = jnp.sum((sn * x).reshape(rows, S, D), axis=1)
    sc = jnp.tanh(jax.lax.dot(c, w1, preferred_element_type=jnp.float32))
    t = pn + sc * c
    agg = jax.lax.dot(t, w2, preferred_element_type=jnp.float32)
    m = m_ref[...]
    o_ref[...] = jnp.where(m > 0.0, c, agg)


def _mlp_all_rows(center_feat, neighbor_flat, W1, W2, is_neg, *, interpret=False):
    B, D = center_feat.shape
    S = neighbor_flat.shape[0] // B
    R = _R
    grid = (B // R,)
    body = functools.partial(_mlp_body, rows=R, S=S, D=D)
    return pl.pallas_call(
        body,
        grid=grid,
        in_specs=[
            pl.BlockSpec((R, D), lambda i: (i, 0)),
            pl.BlockSpec((R * S, D), lambda i: (i, 0)),
            pl.BlockSpec((D, D), lambda i: (0, 0)),
            pl.BlockSpec((D, D), lambda i: (0, 0)),
            pl.BlockSpec((R, 1), lambda i: (i, 0)),
        ],
        out_specs=pl.BlockSpec((R, D), lambda i: (i, 0)),
        out_shape=jax.ShapeDtypeStruct((B, D), jnp.float32),
        compiler_params=pltpu.CompilerParams(
            dimension_semantics=("parallel",),
        ),
        interpret=interpret,
    )(center_feat, neighbor_flat, W1, W2, is_neg)


def kernel(center_feat, neighbor_feats, W1, W2):
    B, D = center_feat.shape
    S = neighbor_feats.shape[1]
    ano = int(B * 0.1)
    batch_center = jnp.mean(center_feat, axis=-1)
    diff_center = jnp.sum(center_feat - batch_center[:, None], axis=-1)
    sorted_idx = jnp.argsort(diff_center)
    neg_idx = sorted_idx[B - ano:]
    is_neg = jnp.zeros((B,), jnp.float32).at[neg_idx].set(1.0)[:, None]
    out = _mlp_all_rows(center_feat, neighbor_feats.reshape(B * S, D), W1, W2,
                        is_neg)
    return out, neg_idx
